# baseline (speedup 1.0000x reference)
import functools

import jax
import jax.numpy as jnp
from jax import lax
from jax.experimental import pallas as pl
from jax.experimental.pallas import tpu as pltpu
from jax.experimental.pallas import tpu_sc as plsc

_PAIR = 2


@functools.lru_cache(maxsize=None)
def _build(B, V, D):
    info = plsc.get_sparse_core_info()
    NC, NS, L = info.num_cores, info.num_subcores, info.num_lanes
    NW = NC * NS
    b_per_w = B // NW
    C = 64
    n_chunks = b_per_w // C
    assert B % (8 * NW) == 0 and D % L == 0 and b_per_w % C == 0
    mesh = plsc.VectorSubcoreMesh(core_axis_name="c", subcore_axis_name="s")

    @functools.partial(
        pl.kernel,
        mesh=mesh,
        out_type=jax.ShapeDtypeStruct((B, D), jnp.float32),
        scratch_types=[
            pltpu.VMEM((b_per_w,), jnp.int32),
            pltpu.VMEM((C,), jnp.int32),
            pltpu.VMEM((C, _PAIR * D), jnp.float32),
            pltpu.VMEM((b_per_w, D), jnp.float32),
            pltpu.SemaphoreType.DMA,
        ],
        compiler_params=pltpu.CompilerParams(needs_layout_passes=False),
    )
    def gather_kernel(emb2_hbm, idx_hbm, out_hbm, idx_v, gidx_v, pairs_v,
                      out_v, sem):
        wid = lax.axis_index("s") * NC + lax.axis_index("c")
        base = wid * b_per_w
        pltpu.sync_copy(idx_hbm.at[pl.ds(base, b_per_w)], idx_v)
        lanes = lax.iota(jnp.int32, L)

        def chunk_body(ci, _):
            for j in range(C // L):
                sl = pl.ds(ci * C + j * L, L)
                gidx_v[pl.ds(j * L, L)] = lax.shift_right_logical(idx_v[sl], 1)
            pltpu.async_copy(emb2_hbm.at[gidx_v], pairs_v, sem).wait()

            def sel16(j, _):
                row0 = ci * C + j * L
                half0 = jnp.bitwise_and(idx_v[pl.ds(row0, L)], 1) * D
                p_vec = j * L + lanes
                rows = row0 + lanes

                def colstep(c, _):
                    csp = jnp.full((L,), 0, jnp.int32) + c
                    vals = plsc.load_gather(pairs_v, [p_vec, half0 + c])
                    plsc.store_scatter(out_v, [rows, csp], vals)
                    return 0

                return lax.fori_loop(0, D, colstep, 0)

            return lax.fori_loop(0, C // L, sel16, 0)

        lax.fori_loop(0, n_chunks, chunk_body, 0)
        pltpu.sync_copy(out_v, out_hbm.at[pl.ds(base, b_per_w)])

    return gather_kernel


def kernel(idx, emb):
    B, = idx.shape
    V, D = emb.shape
    emb2 = emb.reshape(V // _PAIR, _PAIR * D)
    return _build(B, V, D)(emb2, idx.astype(jnp.int32))


# speedup vs baseline: 4.2829x; 4.2829x over previous
import functools

import jax
import jax.numpy as jnp
from jax import lax
from jax.experimental import pallas as pl
from jax.experimental.pallas import tpu as pltpu
from jax.experimental.pallas import tpu_sc as plsc

_W = 8192


@functools.lru_cache(maxsize=None)
def _build(B, V, D):
    info = plsc.get_sparse_core_info()
    NC, NS, L = info.num_cores, info.num_subcores, info.num_lanes
    NW = NC * NS
    FPT = D // NW
    n_full = V // _W
    mid = (V - n_full * _W) // 128 * 128
    tail = V - n_full * _W - mid
    n_pairs = n_full // 2
    assert FPT >= 1 and _W % 128 == 0 and n_full % 2 == 0 and B % L == 0
    mesh = plsc.VectorSubcoreMesh(core_axis_name="c", subcore_axis_name="s")

    @functools.partial(
        pl.kernel,
        mesh=mesh,
        out_type=jax.ShapeDtypeStruct((D, B), jnp.float32),
        scratch_types=[
            pltpu.VMEM((B + L,), jnp.int32),
            pltpu.VMEM((B + L,), jnp.int32),
            pltpu.VMEM((FPT, _W), jnp.float32),
            pltpu.VMEM((FPT, _W), jnp.float32),
            pltpu.VMEM((FPT, max(tail, 1)), jnp.float32),
            pltpu.VMEM((FPT, B), jnp.float32),
            pltpu.SemaphoreType.DMA,
            pltpu.SemaphoreType.DMA,
        ],
        compiler_params=pltpu.CompilerParams(needs_layout_passes=False),
    )
    def gather_kernel(embt_hbm, sidx_hbm, pos_hbm, tailt_hbm, outt_hbm,
                      sidx_v, pos_v, buf0, buf1, tbuf, outt_v, sem0, sem1):
        wid = lax.axis_index("s") * NC + lax.axis_index("c")
        f0 = wid * FPT
        rows = pl.ds(f0, FPT)
        pltpu.sync_copy(sidx_hbm, sidx_v)
        pltpu.sync_copy(pos_hbm, pos_v)

        def proc_block(n0, n1, buf, j):
            def cond(carry):
                return carry[1]

            def step(carry):
                j, _ = carry
                v = sidx_v[pl.ds(j, L)]
                mask = v < n1
                local = v - n0
                p = pos_v[pl.ds(j, L)]
                for f in range(FPT):
                    fs = jnp.full((L,), 0, jnp.int32) + f
                    vals = plsc.load_gather(buf, [fs, local], mask=mask)
                    plsc.store_scatter(outt_v, [fs, p], vals, mask=mask)
                cnt = plsc.all_reduce_population_count(mask)[0]
                return j + cnt, cnt == L

            j, _ = lax.while_loop(cond, step, (j, True))
            return j

        def start(k, buf, sem):
            pltpu.async_copy(
                embt_hbm.at[rows, pl.ds(k * _W, _W)], buf, sem)

        def wait(k, buf, sem):
            pltpu.make_async_copy(
                embt_hbm.at[rows, pl.ds(k * _W, _W)], buf, sem).wait()

        start(0, buf0, sem0)

        def pair_body(i, j):
            k0 = 2 * i
            start(k0 + 1, buf1, sem1)
            wait(k0, buf0, sem0)
            j = proc_block(k0 * _W, (k0 + 1) * _W, buf0, j)

            @pl.when(i < n_pairs - 1)
            def _():
                start(k0 + 2, buf0, sem0)

            wait(k0 + 1, buf1, sem1)
            j = proc_block((k0 + 1) * _W, (k0 + 2) * _W, buf1, j)
            return j

        j = lax.fori_loop(0, n_pairs, pair_body, 0)

        if mid:
            mbuf = buf0.at[:, pl.ds(0, mid)]
            pltpu.sync_copy(embt_hbm.at[rows, pl.ds(n_full * _W, mid)], mbuf)
            j = proc_block(n_full * _W, n_full * _W + mid, mbuf, j)

        if tail:
            pltpu.sync_copy(tailt_hbm.at[rows, :], tbuf)
            proc_block(V - tail, V, tbuf, j)

        pltpu.sync_copy(outt_v, outt_hbm.at[rows, :])

    return gather_kernel


def kernel(idx, emb):
    B, = idx.shape
    V, D = emb.shape
    idx32 = idx.astype(jnp.int32)
    sidx, pos = lax.sort_key_val(idx32, lax.iota(jnp.int32, B))
    sidx = jnp.concatenate([sidx, jnp.full((16,), jnp.int32(2**31 - 1))])
    pos = jnp.concatenate([pos, jnp.zeros((16,), jnp.int32)])
    tail = V % 128
    tailt = emb[V - tail:, :].T if tail else jnp.zeros((D, 1), jnp.float32)
    outt = _build(B, V, D)(emb.T, sidx, pos, tailt)
    return outt.T


# speedup vs baseline: 4.7210x; 1.1023x over previous
import functools

import jax
import jax.numpy as jnp
from jax import lax
from jax.experimental import pallas as pl
from jax.experimental.pallas import tpu as pltpu
from jax.experimental.pallas import tpu_sc as plsc

_W = 4096
_NBUF = 4


@functools.lru_cache(maxsize=None)
def _build(B, V, D):
    info = plsc.get_sparse_core_info()
    NC, NS, L = info.num_cores, info.num_subcores, info.num_lanes
    NW = NC * NS
    FPT = D // NW
    n_full = V // _W
    mid = (V - n_full * _W) // 128 * 128
    tail = V - n_full * _W - mid
    n_grp = n_full // _NBUF
    assert FPT >= 1 and _W % 128 == 0 and n_full % _NBUF == 0 and B % L == 0
    mesh = plsc.VectorSubcoreMesh(core_axis_name="c", subcore_axis_name="s")

    @functools.partial(
        pl.kernel,
        mesh=mesh,
        out_type=jax.ShapeDtypeStruct((D, B), jnp.float32),
        scratch_types=[
            pltpu.VMEM((B + L,), jnp.int32),
            pltpu.VMEM((B + L,), jnp.int32),
            [pltpu.VMEM((FPT, _W), jnp.float32)] * _NBUF,
            pltpu.VMEM((FPT, max(tail, 1)), jnp.float32),
            pltpu.VMEM((FPT, B), jnp.float32),
            [pltpu.SemaphoreType.DMA] * _NBUF,
        ],
        compiler_params=pltpu.CompilerParams(needs_layout_passes=False),
    )
    def gather_kernel(embt_hbm, sidx_hbm, pos_hbm, tailt_hbm, outt_hbm,
                      sidx_v, pos_v, bufs, tbuf, outt_v, sems):
        wid = lax.axis_index("s") * NC + lax.axis_index("c")
        f0 = wid * FPT
        rows = pl.ds(f0, FPT)
        pltpu.sync_copy(sidx_hbm, sidx_v)
        pltpu.sync_copy(pos_hbm, pos_v)

        def proc_block(n0, n1, buf, j):
            def cond(carry):
                return carry[1]

            def step(carry):
                j, _ = carry
                v = sidx_v[pl.ds(j, L)]
                mask = v < n1
                local = v - n0
                p = pos_v[pl.ds(j, L)]
                for f in range(FPT):
                    fs = jnp.full((L,), 0, jnp.int32) + f
                    vals = plsc.load_gather(buf, [fs, local], mask=mask)
                    plsc.store_scatter(outt_v, [fs, p], vals, mask=mask)
                cnt = plsc.all_reduce_population_count(mask)[0]
                return j + cnt, cnt == L

            j, _ = lax.while_loop(cond, step, (j, True))
            return j

        def start(k, buf, sem):
            pltpu.async_copy(
                embt_hbm.at[rows, pl.ds(k * _W, _W)], buf, sem)

        def wait(k, buf, sem):
            pltpu.make_async_copy(
                embt_hbm.at[rows, pl.ds(k * _W, _W)], buf, sem).wait()

        for u in range(_NBUF):
            start(u, bufs[u], sems[u])

        def grp_body(i, j):
            k0 = _NBUF * i
            for u in range(_NBUF):
                k = k0 + u
                wait(k, bufs[u], sems[u])
                j = proc_block(k * _W, (k + 1) * _W, bufs[u], j)

                @pl.when(i < n_grp - 1)
                def _():
                    start(k + _NBUF, bufs[u], sems[u])

            return j

        j = lax.fori_loop(0, n_grp, grp_body, 0)

        if mid:
            mbuf = bufs[0].at[:, pl.ds(0, mid)]
            pltpu.sync_copy(embt_hbm.at[rows, pl.ds(n_full * _W, mid)], mbuf)
            j = proc_block(n_full * _W, n_full * _W + mid, mbuf, j)

        if tail:
            pltpu.sync_copy(tailt_hbm.at[rows, :], tbuf)
            proc_block(V - tail, V, tbuf, j)

        pltpu.sync_copy(outt_v, outt_hbm.at[rows, :])

    return gather_kernel


def kernel(idx, emb):
    B, = idx.shape
    V, D = emb.shape
    idx32 = idx.astype(jnp.int32)
    sidx, pos = lax.sort_key_val(idx32, lax.iota(jnp.int32, B))
    sidx = jnp.concatenate([sidx, jnp.full((16,), jnp.int32(2**31 - 1))])
    pos = jnp.concatenate([pos, jnp.zeros((16,), jnp.int32)])
    tail = V % 128
    tailt = emb[V - tail:, :].T if tail else jnp.zeros((D, 1), jnp.float32)
    outt = _build(B, V, D)(emb.T, sidx, pos, tailt)
    return outt.T


# speedup vs baseline: 5.1180x; 1.0841x over previous
import functools

import jax
import jax.numpy as jnp
from jax import lax
from jax.experimental import pallas as pl
from jax.experimental.pallas import tpu as pltpu
from jax.experimental.pallas import tpu_sc as plsc

_W = 2048
_NBUF = 8


@functools.lru_cache(maxsize=None)
def _build(B, V, D):
    info = plsc.get_sparse_core_info()
    NC, NS, L = info.num_cores, info.num_subcores, info.num_lanes
    NW = NC * NS
    FPT = D // NW
    n_full = V // _W
    mid = (V - n_full * _W) // 128 * 128
    tail = V - n_full * _W - mid
    n_grp = n_full // _NBUF
    assert FPT >= 1 and _W % 128 == 0 and n_full % _NBUF == 0 and B % L == 0
    mesh = plsc.VectorSubcoreMesh(core_axis_name="c", subcore_axis_name="s")

    @functools.partial(
        pl.kernel,
        mesh=mesh,
        out_type=jax.ShapeDtypeStruct((D, B), jnp.float32),
        scratch_types=[
            pltpu.VMEM((B + L,), jnp.int32),
            pltpu.VMEM((B + L,), jnp.int32),
            [pltpu.VMEM((FPT, _W), jnp.float32)] * _NBUF,
            pltpu.VMEM((FPT, max(tail, 1)), jnp.float32),
            pltpu.VMEM((FPT, B), jnp.float32),
            [pltpu.SemaphoreType.DMA] * _NBUF,
        ],
        compiler_params=pltpu.CompilerParams(needs_layout_passes=False),
    )
    def gather_kernel(embt_hbm, sidx_hbm, pos_hbm, tailt_hbm, outt_hbm,
                      sidx_v, pos_v, bufs, tbuf, outt_v, sems):
        wid = lax.axis_index("s") * NC + lax.axis_index("c")
        f0 = wid * FPT
        rows = pl.ds(f0, FPT)
        pltpu.sync_copy(sidx_hbm, sidx_v)
        pltpu.sync_copy(pos_hbm, pos_v)

        def proc_block(n0, n1, buf, j):
            def cond(carry):
                return carry[1]

            def step(carry):
                j, _ = carry
                v = sidx_v[pl.ds(j, L)]
                mask = v < n1
                local = v - n0
                p = pos_v[pl.ds(j, L)]
                for f in range(FPT):
                    fs = jnp.full((L,), 0, jnp.int32) + f
                    vals = plsc.load_gather(buf, [fs, local], mask=mask)
                    plsc.store_scatter(outt_v, [fs, p], vals, mask=mask)
                cnt = plsc.all_reduce_population_count(mask)[0]
                return j + cnt, cnt == L

            j, _ = lax.while_loop(cond, step, (j, True))
            return j

        def start(k, buf, sem):
            pltpu.async_copy(
                embt_hbm.at[rows, pl.ds(k * _W, _W)], buf, sem)

        def wait(k, buf, sem):
            pltpu.make_async_copy(
                embt_hbm.at[rows, pl.ds(k * _W, _W)], buf, sem).wait()

        for u in range(_NBUF):
            start(u, bufs[u], sems[u])

        def grp_body(i, j):
            k0 = _NBUF * i
            for u in range(_NBUF):
                k = k0 + u
                wait(k, bufs[u], sems[u])
                j = proc_block(k * _W, (k + 1) * _W, bufs[u], j)

                @pl.when(i < n_grp - 1)
                def _():
                    start(k + _NBUF, bufs[u], sems[u])

            return j

        j = lax.fori_loop(0, n_grp, grp_body, 0)

        if mid:
            mbuf = bufs[0].at[:, pl.ds(0, mid)]
            pltpu.sync_copy(embt_hbm.at[rows, pl.ds(n_full * _W, mid)], mbuf)
            j = proc_block(n_full * _W, n_full * _W + mid, mbuf, j)

        if tail:
            pltpu.sync_copy(tailt_hbm.at[rows, :], tbuf)
            proc_block(V - tail, V, tbuf, j)

        pltpu.sync_copy(outt_v, outt_hbm.at[rows, :])

    return gather_kernel


def kernel(idx, emb):
    B, = idx.shape
    V, D = emb.shape
    idx32 = idx.astype(jnp.int32)
    sidx, pos = lax.sort_key_val(idx32, lax.iota(jnp.int32, B))
    sidx = jnp.concatenate([sidx, jnp.full((16,), jnp.int32(2**31 - 1))])
    pos = jnp.concatenate([pos, jnp.zeros((16,), jnp.int32)])
    tail = V % 128
    tailt = emb[V - tail:, :].T if tail else jnp.zeros((D, 1), jnp.float32)
    outt = _build(B, V, D)(emb.T, sidx, pos, tailt)
    return outt.T
